# baseline (speedup 1.0000x reference)
import functools

import jax
import jax.numpy as jnp
from jax import lax
from jax.experimental import pallas as pl
from jax.experimental.pallas import tpu as pltpu
from jax.experimental.pallas import tpu_sc as plsc

N = 10000
E = 320000
H = 128
OUT = 128
NUM_GRAPHS = 64
MAX_ITER = 10
M_PARAM = 0.5
ATOM_FEATS = 9
ATOM_VOCAB = 128

NB = 10
BN = N // NB

NW = 32
EW = E // NW
C = 80
NCH = EW // C
GC = 25
NST = NCH // GC
RPT = 624
TAIL = N - 16 * RPT

_HIGH = lax.Precision.HIGHEST


def _gelu(x):
    return 0.5 * x * (1.0 + lax.erf(x * 0.7071067811865476))


def _enc_body(data_ref, emb_ref, w1_ref, b1_ref, w2_ref, b2_ref, h_ref, z_ref):
    d = data_ref[...]
    lanes = lax.broadcasted_iota(jnp.int32, (1, ATOM_VOCAB), 1)
    acc = jnp.zeros((BN, H), jnp.float32)
    for f in range(ATOM_FEATS):
        onehot = (d[:, f:f + 1] == lanes).astype(jnp.float32)
        acc = acc + jnp.dot(onehot, emb_ref[f], precision=_HIGH,
                            preferred_element_type=jnp.float32)
    acc = jnp.maximum(acc, 0.0)
    g = _gelu(jnp.dot(acc, w1_ref[...], precision=_HIGH,
                      preferred_element_type=jnp.float32) + b1_ref[...])
    henc = jnp.dot(g, w2_ref[...], precision=_HIGH,
                   preferred_element_type=jnp.float32) + b2_ref[...]
    h_ref[...] = henc
    z_ref[...] = jnp.maximum(henc, 0.0)


def _encoder(data, atom_emb, w1, b1, w2, b2):
    return pl.pallas_call(
        _enc_body,
        grid=(NB,),
        in_specs=[
            pl.BlockSpec((BN, ATOM_FEATS), lambda i: (i, 0)),
            pl.BlockSpec((ATOM_FEATS, ATOM_VOCAB, H), lambda i: (0, 0, 0)),
            pl.BlockSpec((H, H), lambda i: (0, 0)),
            pl.BlockSpec((1, H), lambda i: (0, 0)),
            pl.BlockSpec((H, H), lambda i: (0, 0)),
            pl.BlockSpec((1, H), lambda i: (0, 0)),
        ],
        out_specs=[pl.BlockSpec((BN, H), lambda i: (i, 0)),
                   pl.BlockSpec((BN, H), lambda i: (i, 0))],
        out_shape=[jax.ShapeDtypeStruct((N, H), jnp.float32),
                   jax.ShapeDtypeStruct((N, H), jnp.float32)],
    )(data, atom_emb, w1, b1, w2, b2)


def _wt_body(r_ref, s_ref, wt_ref):
    Rm = r_ref[...]
    Sm = s_ref[...]
    rrT = lax.dot_general(Rm, Rm, (((1,), (1,)), ((), ())), precision=_HIGH,
                          preferred_element_type=jnp.float32)
    i0 = lax.broadcasted_iota(jnp.int32, (H, H), 0)
    i1 = lax.broadcasted_iota(jnp.int32, (H, H), 1)
    eye = (i0 == i1).astype(jnp.float32)
    sT = lax.dot_general(Sm, eye, (((0,), (0,)), ((), ())), precision=_HIGH,
                         preferred_element_type=jnp.float32)
    wt_ref[...] = (1.0 - M_PARAM) * eye - rrT + (sT - Sm)


def _wt(Rm, Sm):
    return pl.pallas_call(
        _wt_body,
        in_specs=[pl.BlockSpec((H, H), lambda: (0, 0)),
                  pl.BlockSpec((H, H), lambda: (0, 0))],
        out_specs=pl.BlockSpec((H, H), lambda: (0, 0)),
        out_shape=jax.ShapeDtypeStruct((H, H), jnp.float32),
    )(Rm, Sm)


def _make_spmm(width):
    nvr = width // 16
    mesh = plsc.VectorSubcoreMesh(core_axis_name="c", subcore_axis_name="s")

    @functools.partial(
        pl.kernel,
        out_type=jax.ShapeDtypeStruct((2, N, width), jnp.float32),
        mesh=mesh,
        scratch_types=[
            pltpu.VMEM((GC, C), jnp.int32),
            pltpu.VMEM((GC, C), jnp.int32),
            pltpu.VMEM((GC, C), jnp.float32),
            pltpu.VMEM((C, width), jnp.float32),
            pltpu.VMEM((C, width), jnp.float32),
            pltpu.VMEM_SHARED((N, width), jnp.float32),
            pltpu.SemaphoreType.DMA,
            pltpu.SemaphoreType.DMA,
        ],
    )
    def spmm(z_hbm, src_hbm, dst_hbm, w_hbm, zero_hbm, out_hbm,
             src_v, dst_v, w_v, rows0, rows1, agg_s, sem0, sem1):
        cid = lax.axis_index("c")
        sid = lax.axis_index("s")
        wid = sid * 2 + cid

        pltpu.sync_copy(zero_hbm.at[pl.ds(sid * RPT, RPT)],
                        agg_s.at[pl.ds(sid * RPT, RPT)])

        @pl.when(sid == 15)
        def _zero_tail():
            pltpu.sync_copy(zero_hbm.at[pl.ds(16 * RPT, TAIL)],
                            agg_s.at[pl.ds(16 * RPT, TAIL)])

        plsc.subcore_barrier()

        def scale_scatter(ci, rows_v):
            def group_body(g, c2):
                wv = w_v[ci, pl.ds(g * 16, 16)]
                for k in range(16):
                    w = wv[k]
                    i = g * 16 + k
                    for j in range(nvr):
                        sl = pl.ds(j * 16, 16)
                        rows_v[i, sl] = rows_v[i, sl] * w
                return c2

            lax.fori_loop(0, C // 16, group_body, 0)
            pltpu.sync_copy(rows_v, agg_s.at[dst_v.at[ci]], add=True)

        def stage_body(s, carry0):
            pltpu.sync_copy(src_hbm.at[wid, s], src_v)
            pltpu.sync_copy(dst_hbm.at[wid, s], dst_v)
            pltpu.sync_copy(w_hbm.at[wid, s], w_v)
            pltpu.async_copy(z_hbm.at[src_v.at[0]], rows0, sem0)

            def pair_body(p, carry):
                c0 = 2 * p
                c1 = c0 + 1
                c2 = c0 + 2

                @pl.when(c1 < GC)
                def _g1():
                    pltpu.async_copy(z_hbm.at[src_v.at[c1]], rows1, sem1)

                pltpu.make_async_copy(z_hbm.at[src_v.at[c0]], rows0,
                                      sem0).wait()
                scale_scatter(c0, rows0)

                @pl.when(c2 < GC)
                def _g2():
                    pltpu.async_copy(z_hbm.at[src_v.at[c2]], rows0, sem0)

                @pl.when(c1 < GC)
                def _p1():
                    pltpu.make_async_copy(z_hbm.at[src_v.at[c1]], rows1,
                                          sem1).wait()
                    scale_scatter(c1, rows1)

                return carry

            lax.fori_loop(0, (GC + 1) // 2, pair_body, 0)
            return carry0

        lax.fori_loop(0, NST, stage_body, 0)
        plsc.subcore_barrier()
        pltpu.sync_copy(agg_s.at[pl.ds(sid * RPT, RPT)],
                        out_hbm.at[cid, pl.ds(sid * RPT, RPT)])

        @pl.when(sid == 15)
        def _write_tail():
            pltpu.sync_copy(agg_s.at[pl.ds(16 * RPT, TAIL)],
                            out_hbm.at[cid, pl.ds(16 * RPT, TAIL)])

    return spmm


_spmm128 = _make_spmm(H)


DW = 128

_deg_mesh = plsc.VectorSubcoreMesh(core_axis_name="c", subcore_axis_name="s")


@functools.partial(
    pl.kernel,
    out_type=jax.ShapeDtypeStruct((2, N, DW), jnp.float32),
    mesh=_deg_mesh,
    scratch_types=[
        pltpu.VMEM((GC, C), jnp.int32),
        pltpu.VMEM((GC, C), jnp.float32),
        pltpu.VMEM((C, DW), jnp.float32),
        pltpu.VMEM_SHARED((N, DW), jnp.float32),
    ],
)
def _deg_kernel(dst_hbm, w_hbm, zero_hbm, out_hbm, dst_v, w_v, rows_v, acc_s):
    cid = lax.axis_index("c")
    sid = lax.axis_index("s")
    wid = sid * 2 + cid

    pltpu.sync_copy(zero_hbm.at[pl.ds(sid * RPT, RPT)],
                    acc_s.at[pl.ds(sid * RPT, RPT)])

    @pl.when(sid == 15)
    def _zero_tail():
        pltpu.sync_copy(zero_hbm.at[pl.ds(16 * RPT, TAIL)],
                        acc_s.at[pl.ds(16 * RPT, TAIL)])

    pltpu.sync_copy(zero_hbm.at[pl.ds(0, C)], rows_v)
    plsc.subcore_barrier()

    def stage_body(s, carry0):
        pltpu.sync_copy(dst_hbm.at[wid, s], dst_v)
        pltpu.sync_copy(w_hbm.at[wid, s], w_v)

        def chunk_body(ci, carry):
            def group_body(g, c2):
                wv = w_v[ci, pl.ds(g * 16, 16)]
                for k in range(16):
                    i = g * 16 + k
                    sl = pl.ds(0, 16)
                    rows_v[i, sl] = rows_v[i, sl] * 0.0 + wv[k]
                return c2

            lax.fori_loop(0, C // 16, group_body, 0)
            pltpu.sync_copy(rows_v, acc_s.at[dst_v.at[ci]], add=True)
            return carry

        lax.fori_loop(0, GC, chunk_body, 0)
        return carry0

    lax.fori_loop(0, NST, stage_body, 0)
    plsc.subcore_barrier()
    pltpu.sync_copy(acc_s.at[pl.ds(sid * RPT, RPT)],
                    out_hbm.at[cid, pl.ds(sid * RPT, RPT)])

    @pl.when(sid == 15)
    def _write_tail():
        pltpu.sync_copy(acc_s.at[pl.ds(16 * RPT, TAIL)],
                        out_hbm.at[cid, pl.ds(16 * RPT, TAIL)])


def _step_body(agg_ref, deg_ref, h_ref, wt_ref, z_ref):
    a = agg_ref[0] + agg_ref[1]
    deg = deg_ref[0] + deg_ref[1]
    a = a * (1.0 / jnp.maximum(deg, 1e-6))
    z = jnp.dot(a, wt_ref[...], precision=_HIGH,
                preferred_element_type=jnp.float32) + h_ref[...]
    z_ref[...] = jnp.maximum(z, 0.0)


def _dense_step(aggp, degp, henc, wt):
    return pl.pallas_call(
        _step_body,
        grid=(NB,),
        in_specs=[
            pl.BlockSpec((2, BN, H), lambda i: (0, i, 0)),
            pl.BlockSpec((2, BN, 1), lambda i: (0, i, 0)),
            pl.BlockSpec((BN, H), lambda i: (i, 0)),
            pl.BlockSpec((H, H), lambda i: (0, 0)),
        ],
        out_specs=pl.BlockSpec((BN, H), lambda i: (i, 0)),
        out_shape=jax.ShapeDtypeStruct((N, H), jnp.float32),
    )(aggp, degp, henc, wt)


def _final_body(z_ref, batch_ref, gamma_ref, beta_ref,
                wd1_ref, bd1_ref, wd2_ref, bd2_ref, out_ref):
    hfull = z_ref[...]
    mean = jnp.sum(hfull, axis=0, keepdims=True) * (1.0 / N)
    m2 = jnp.sum(hfull * hfull, axis=0, keepdims=True) * (1.0 / N)
    var = m2 - mean * mean
    hn = (hfull - mean) * lax.rsqrt(var + 1e-5) * gamma_ref[...] + beta_ref[...]
    hn = jnp.maximum(hn, 0.0)
    b = batch_ref[...]
    gids = lax.broadcasted_iota(jnp.int32, (NUM_GRAPHS, N), 0)
    onehot = (b == gids).astype(jnp.float32)
    sums = jnp.dot(onehot, hn, precision=_HIGH,
                   preferred_element_type=jnp.float32)
    cnts = jnp.sum(onehot, axis=1, keepdims=True)
    pooled = sums * (1.0 / jnp.maximum(cnts, 1.0))
    g = _gelu(jnp.dot(pooled, wd1_ref[...], precision=_HIGH,
                      preferred_element_type=jnp.float32) + bd1_ref[...])
    out_ref[...] = jnp.dot(g, wd2_ref[...], precision=_HIGH,
                           preferred_element_type=jnp.float32) + bd2_ref[...]


def _final(z, batch2d, gamma, beta, wd1, bd1, wd2, bd2):
    return pl.pallas_call(
        _final_body,
        in_specs=[
            pl.BlockSpec((N, H), lambda: (0, 0)),
            pl.BlockSpec((1, N), lambda: (0, 0)),
            pl.BlockSpec((1, H), lambda: (0, 0)),
            pl.BlockSpec((1, H), lambda: (0, 0)),
            pl.BlockSpec((H, H), lambda: (0, 0)),
            pl.BlockSpec((1, H), lambda: (0, 0)),
            pl.BlockSpec((H, OUT), lambda: (0, 0)),
            pl.BlockSpec((1, OUT), lambda: (0, 0)),
        ],
        out_specs=pl.BlockSpec((NUM_GRAPHS, OUT), lambda: (0, 0)),
        out_shape=jax.ShapeDtypeStruct((NUM_GRAPHS, OUT), jnp.float32),
    )(z, batch2d, gamma, beta, wd1, bd1, wd2, bd2)


def kernel(data, x, edge_index, edge_weight, batch, atom_emb, W_enc1, b_enc1,
           W_enc2, b_enc2, R, S, bn_gamma, bn_beta, W_dec1, b_dec1, W_dec2,
           b_dec2):
    src = edge_index[0].reshape(NW, NST, GC, C)
    dst = edge_index[1].reshape(NW, NST, GC, C)
    ew = edge_weight.reshape(NW, NST, GC, C)

    henc, zt = _encoder(data.astype(jnp.int32), atom_emb,
                        W_enc1, b_enc1.reshape(1, H),
                        W_enc2, b_enc2.reshape(1, H))
    wt = _wt(R, S)

    zeros128 = jnp.zeros((N, H), jnp.float32)
    degp = _deg_kernel(dst, ew, zeros128)
    degp = degp[:, :, 0:1]

    for _ in range(MAX_ITER - 1):
        aggp = _spmm128(zt, src, dst, ew, zeros128)
        zt = _dense_step(aggp, degp, henc, wt)

    return _final(zt, batch.reshape(1, N).astype(jnp.int32),
                  bn_gamma.reshape(1, H), bn_beta.reshape(1, H),
                  W_dec1, b_dec1.reshape(1, OUT), W_dec2,
                  b_dec2.reshape(1, OUT))


# speedup vs baseline: 1.1320x; 1.1320x over previous
import functools

import jax
import jax.numpy as jnp
from jax import lax
from jax.experimental import pallas as pl
from jax.experimental.pallas import tpu as pltpu
from jax.experimental.pallas import tpu_sc as plsc

N = 10000
E = 320000
H = 128
OUT = 128
NUM_GRAPHS = 64
MAX_ITER = 10
M_PARAM = 0.5
ATOM_FEATS = 9
ATOM_VOCAB = 128

NB = 10
BN = N // NB

NW = 32
EW = E // NW
C = 80
NCH = EW // C
GC = 25
NST = NCH // GC
RPT = 624
TAIL = N - 16 * RPT

_HIGH = lax.Precision.HIGHEST


def _gelu(x):
    return 0.5 * x * (1.0 + lax.erf(x * 0.7071067811865476))


def _enc_body(data_ref, emb_ref, w1_ref, b1_ref, w2_ref, b2_ref, h_ref, z_ref):
    d = data_ref[...]
    lanes = lax.broadcasted_iota(jnp.int32, (1, ATOM_VOCAB), 1)
    acc = jnp.zeros((BN, H), jnp.float32)
    for f in range(ATOM_FEATS):
        onehot = (d[:, f:f + 1] == lanes).astype(jnp.float32)
        acc = acc + jnp.dot(onehot, emb_ref[f], precision=_HIGH,
                            preferred_element_type=jnp.float32)
    acc = jnp.maximum(acc, 0.0)
    g = _gelu(jnp.dot(acc, w1_ref[...], precision=_HIGH,
                      preferred_element_type=jnp.float32) + b1_ref[...])
    henc = jnp.dot(g, w2_ref[...], precision=_HIGH,
                   preferred_element_type=jnp.float32) + b2_ref[...]
    h_ref[...] = henc
    z_ref[...] = jnp.maximum(henc, 0.0)


def _encoder(data, atom_emb, w1, b1, w2, b2):
    return pl.pallas_call(
        _enc_body,
        grid=(NB,),
        in_specs=[
            pl.BlockSpec((BN, ATOM_FEATS), lambda i: (i, 0)),
            pl.BlockSpec((ATOM_FEATS, ATOM_VOCAB, H), lambda i: (0, 0, 0)),
            pl.BlockSpec((H, H), lambda i: (0, 0)),
            pl.BlockSpec((1, H), lambda i: (0, 0)),
            pl.BlockSpec((H, H), lambda i: (0, 0)),
            pl.BlockSpec((1, H), lambda i: (0, 0)),
        ],
        out_specs=[pl.BlockSpec((BN, H), lambda i: (i, 0)),
                   pl.BlockSpec((BN, H), lambda i: (i, 0))],
        out_shape=[jax.ShapeDtypeStruct((N, H), jnp.float32),
                   jax.ShapeDtypeStruct((N, H), jnp.float32)],
    )(data, atom_emb, w1, b1, w2, b2)


def _wt_body(r_ref, s_ref, wt_ref):
    Rm = r_ref[...]
    Sm = s_ref[...]
    rrT = lax.dot_general(Rm, Rm, (((1,), (1,)), ((), ())), precision=_HIGH,
                          preferred_element_type=jnp.float32)
    i0 = lax.broadcasted_iota(jnp.int32, (H, H), 0)
    i1 = lax.broadcasted_iota(jnp.int32, (H, H), 1)
    eye = (i0 == i1).astype(jnp.float32)
    sT = lax.dot_general(Sm, eye, (((0,), (0,)), ((), ())), precision=_HIGH,
                         preferred_element_type=jnp.float32)
    wt_ref[...] = (1.0 - M_PARAM) * eye - rrT + (sT - Sm)


def _wt(Rm, Sm):
    return pl.pallas_call(
        _wt_body,
        in_specs=[pl.BlockSpec((H, H), lambda: (0, 0)),
                  pl.BlockSpec((H, H), lambda: (0, 0))],
        out_specs=pl.BlockSpec((H, H), lambda: (0, 0)),
        out_shape=jax.ShapeDtypeStruct((H, H), jnp.float32),
    )(Rm, Sm)


def _make_spmm(width):
    nvr = width // 16
    mesh = plsc.VectorSubcoreMesh(core_axis_name="c", subcore_axis_name="s")

    @functools.partial(
        pl.kernel,
        out_type=jax.ShapeDtypeStruct((2, N, width), jnp.float32),
        mesh=mesh,
        scratch_types=[
            pltpu.VMEM((GC, C), jnp.int32),
            pltpu.VMEM((GC, C), jnp.int32),
            pltpu.VMEM((GC, C), jnp.float32),
            pltpu.VMEM((C, width), jnp.float32),
            pltpu.VMEM((C, width), jnp.float32),
            pltpu.VMEM((C, width), jnp.float32),
            pltpu.VMEM_SHARED((N, width), jnp.float32),
            pltpu.SemaphoreType.DMA,
            pltpu.SemaphoreType.DMA,
            pltpu.SemaphoreType.DMA,
            pltpu.SemaphoreType.DMA,
            pltpu.SemaphoreType.DMA,
            pltpu.SemaphoreType.DMA,
        ],
    )
    def spmm(z_hbm, src_hbm, dst_hbm, w_hbm, zero_hbm, out_hbm,
             src_v, dst_v, w_v, rows0, rows1, rows2, agg_s,
             gs0, gs1, gs2, ss0, ss1, ss2):
        cid = lax.axis_index("c")
        sid = lax.axis_index("s")
        wid = sid * 2 + cid

        pltpu.sync_copy(zero_hbm.at[pl.ds(sid * RPT, RPT)],
                        agg_s.at[pl.ds(sid * RPT, RPT)])

        @pl.when(sid == 15)
        def _zero_tail():
            pltpu.sync_copy(zero_hbm.at[pl.ds(16 * RPT, TAIL)],
                            agg_s.at[pl.ds(16 * RPT, TAIL)])

        plsc.subcore_barrier()

        def scale(ci, rows_v):
            def group_body(g, c2):
                wv = w_v[ci, pl.ds(g * 16, 16)]
                for k in range(16):
                    w = wv[k]
                    i = g * 16 + k
                    for j in range(nvr):
                        sl = pl.ds(j * 16, 16)
                        rows_v[i, sl] = rows_v[i, sl] * w
                return c2

            lax.fori_loop(0, C // 16, group_body, 0)

        def gwait(ci, rows_v, gsem):
            pltpu.make_async_copy(z_hbm.at[src_v.at[ci]], rows_v, gsem).wait()

        def swait(ci, rows_v, ssem):
            pltpu.make_async_copy(rows_v, agg_s.at[dst_v.at[ci]], ssem).wait()

        def stage_body(s, carry0):
            pltpu.sync_copy(src_hbm.at[wid, s], src_v)
            pltpu.sync_copy(dst_hbm.at[wid, s], dst_v)
            pltpu.sync_copy(w_hbm.at[wid, s], w_v)
            pltpu.async_copy(z_hbm.at[src_v.at[0]], rows0, gs0)
            pltpu.async_copy(z_hbm.at[src_v.at[1]], rows1, gs1)
            pltpu.async_copy(z_hbm.at[src_v.at[2]], rows2, gs2)

            def triple_body(t, carry):
                c0 = 3 * t
                c1 = c0 + 1
                c2 = c0 + 2
                n0 = c0 + 3
                n1 = c0 + 4

                gwait(c0, rows0, gs0)
                scale(c0, rows0)
                pltpu.async_copy(rows0, agg_s.at[dst_v.at[c0]], ss0,
                                 add=True)

                @pl.when(t > 0)
                def _turn2():
                    swait(c0 - 1, rows2, ss2)

                    @pl.when(c2 < GC)
                    def _g2():
                        pltpu.async_copy(z_hbm.at[src_v.at[c2]], rows2, gs2)

                @pl.when(c1 < GC)
                def _do1():
                    gwait(c1, rows1, gs1)
                    scale(c1, rows1)
                    pltpu.async_copy(rows1, agg_s.at[dst_v.at[c1]], ss1,
                                     add=True)

                swait(c0, rows0, ss0)

                @pl.when(n0 < GC)
                def _g0():
                    pltpu.async_copy(z_hbm.at[src_v.at[n0]], rows0, gs0)

                @pl.when(c2 < GC)
                def _do2():
                    gwait(c2, rows2, gs2)
                    scale(c2, rows2)
                    pltpu.async_copy(rows2, agg_s.at[dst_v.at[c2]], ss2,
                                     add=True)

                @pl.when(c1 < GC)
                def _turn1():
                    swait(c1, rows1, ss1)

                    @pl.when(n1 < GC)
                    def _g1():
                        pltpu.async_copy(z_hbm.at[src_v.at[n1]], rows1, gs1)

                return carry

            lax.fori_loop(0, (GC + 2) // 3, triple_body, 0)
            return carry0

        lax.fori_loop(0, NST, stage_body, 0)
        plsc.subcore_barrier()
        pltpu.sync_copy(agg_s.at[pl.ds(sid * RPT, RPT)],
                        out_hbm.at[cid, pl.ds(sid * RPT, RPT)])

        @pl.when(sid == 15)
        def _write_tail():
            pltpu.sync_copy(agg_s.at[pl.ds(16 * RPT, TAIL)],
                            out_hbm.at[cid, pl.ds(16 * RPT, TAIL)])

    return spmm


_spmm128 = _make_spmm(H)


def _step_body(agg_ref, deg_ref, h_ref, wt_ref, z_ref):
    a = agg_ref[0] + agg_ref[1]
    deg = deg_ref[0] + deg_ref[1]
    a = a * (1.0 / jnp.maximum(deg, 1e-6))
    z = jnp.dot(a, wt_ref[...], precision=_HIGH,
                preferred_element_type=jnp.float32) + h_ref[...]
    z_ref[...] = jnp.maximum(z, 0.0)


def _dense_step(aggp, degp, henc, wt):
    return pl.pallas_call(
        _step_body,
        grid=(NB,),
        in_specs=[
            pl.BlockSpec((2, BN, H), lambda i: (0, i, 0)),
            pl.BlockSpec((2, BN, 1), lambda i: (0, i, 0)),
            pl.BlockSpec((BN, H), lambda i: (i, 0)),
            pl.BlockSpec((H, H), lambda i: (0, 0)),
        ],
        out_specs=pl.BlockSpec((BN, H), lambda i: (i, 0)),
        out_shape=jax.ShapeDtypeStruct((N, H), jnp.float32),
    )(aggp, degp, henc, wt)


def _final_body(z_ref, batch_ref, gamma_ref, beta_ref,
                wd1_ref, bd1_ref, wd2_ref, bd2_ref, out_ref):
    hfull = z_ref[...]
    mean = jnp.sum(hfull, axis=0, keepdims=True) * (1.0 / N)
    m2 = jnp.sum(hfull * hfull, axis=0, keepdims=True) * (1.0 / N)
    var = m2 - mean * mean
    hn = (hfull - mean) * lax.rsqrt(var + 1e-5) * gamma_ref[...] + beta_ref[...]
    hn = jnp.maximum(hn, 0.0)
    b = batch_ref[...]
    gids = lax.broadcasted_iota(jnp.int32, (NUM_GRAPHS, N), 0)
    onehot = (b == gids).astype(jnp.float32)
    sums = jnp.dot(onehot, hn, precision=_HIGH,
                   preferred_element_type=jnp.float32)
    cnts = jnp.sum(onehot, axis=1, keepdims=True)
    pooled = sums * (1.0 / jnp.maximum(cnts, 1.0))
    g = _gelu(jnp.dot(pooled, wd1_ref[...], precision=_HIGH,
                      preferred_element_type=jnp.float32) + bd1_ref[...])
    out_ref[...] = jnp.dot(g, wd2_ref[...], precision=_HIGH,
                           preferred_element_type=jnp.float32) + bd2_ref[...]


def _final(z, batch2d, gamma, beta, wd1, bd1, wd2, bd2):
    return pl.pallas_call(
        _final_body,
        in_specs=[
            pl.BlockSpec((N, H), lambda: (0, 0)),
            pl.BlockSpec((1, N), lambda: (0, 0)),
            pl.BlockSpec((1, H), lambda: (0, 0)),
            pl.BlockSpec((1, H), lambda: (0, 0)),
            pl.BlockSpec((H, H), lambda: (0, 0)),
            pl.BlockSpec((1, H), lambda: (0, 0)),
            pl.BlockSpec((H, OUT), lambda: (0, 0)),
            pl.BlockSpec((1, OUT), lambda: (0, 0)),
        ],
        out_specs=pl.BlockSpec((NUM_GRAPHS, OUT), lambda: (0, 0)),
        out_shape=jax.ShapeDtypeStruct((NUM_GRAPHS, OUT), jnp.float32),
    )(z, batch2d, gamma, beta, wd1, bd1, wd2, bd2)


def kernel(data, x, edge_index, edge_weight, batch, atom_emb, W_enc1, b_enc1,
           W_enc2, b_enc2, R, S, bn_gamma, bn_beta, W_dec1, b_dec1, W_dec2,
           b_dec2):
    src = edge_index[0].reshape(NW, NST, GC, C)
    dst = edge_index[1].reshape(NW, NST, GC, C)
    ew = edge_weight.reshape(NW, NST, GC, C)

    henc, zt = _encoder(data.astype(jnp.int32), atom_emb,
                        W_enc1, b_enc1.reshape(1, H),
                        W_enc2, b_enc2.reshape(1, H))
    wt = _wt(R, S)

    ones128 = jnp.ones((N, H), jnp.float32)
    zeros128 = jnp.zeros((N, H), jnp.float32)
    degp = _spmm128(ones128, src, dst, ew, zeros128)
    degp = degp[:, :, 0:1]

    for _ in range(MAX_ITER - 1):
        aggp = _spmm128(zt, src, dst, ew, zeros128)
        zt = _dense_step(aggp, degp, henc, wt)

    return _final(zt, batch.reshape(1, N).astype(jnp.int32),
                  bn_gamma.reshape(1, H), bn_beta.reshape(1, H),
                  W_dec1, b_dec1.reshape(1, OUT), W_dec2,
                  b_dec2.reshape(1, OUT))
